# baseline (speedup 1.0000x reference)
import functools

import jax
import jax.numpy as jnp
from jax import lax
from jax.experimental import pallas as pl
from jax.experimental.pallas import tpu as pltpu
from jax.experimental.pallas import tpu_sc as plsc

_NBUF = 5
_LANES = 16
_TPAD = 136


@functools.cache
def _build(b_, l_, d, nw, nc):
    bpw = b_ // nw
    dt_n = d // 8
    mesh = plsc.VectorSubcoreMesh(core_axis_name="c", subcore_axis_name="s")

    @functools.partial(
        pl.kernel,
        mesh=mesh,
        out_type=jax.ShapeDtypeStruct((l_, dt_n, b_ // bpw, 8, bpw), jnp.float32),
        scratch_types=[
            pltpu.VMEM((l_, bpw), jnp.int32),
            *[pltpu.VMEM((bpw, d), jnp.float32) for _ in range(_NBUF)],
            *[pltpu.VMEM((dt_n, 8, _TPAD), jnp.float32) for _ in range(_NBUF)],
            pltpu.SemaphoreType.DMA((_NBUF,)),
            pltpu.SemaphoreType.DMA((_NBUF,)),
        ],
        compiler_params=pltpu.CompilerParams(
            use_tc_tiling_on_sc=False, needs_layout_passes=False),
    )
    def gather_kernel(table, idx_t, out, idx_v, *bufs):
        raws = bufs[:_NBUF]
        tbufs = bufs[_NBUF:2 * _NBUF]
        gsem, osem = bufs[2 * _NBUF], bufs[2 * _NBUF + 1]
        wid = lax.axis_index("s") * nc + lax.axis_index("c")
        b0 = wid * bpw
        pltpu.sync_copy(idx_t.at[:, pl.ds(b0, bpw)], idx_v)

        lane = lax.iota(jnp.int32, _LANES)
        di_idx = lane & 7
        dt_idx = [(lane >> 3) + 2 * q for q in range(d // _LANES)]

        def gather(s, c):
            return pltpu.make_async_copy(
                table.at[idx_v.at[c]], raws[s], gsem.at[s])

        def write(s, c):
            return pltpu.make_async_copy(
                tbufs[s].at[:, :, pl.ds(0, bpw)], out.at[c, :, wid],
                osem.at[s])

        def transpose(s):
            raw, tbuf = raws[s], tbufs[s]

            @plsc.parallel_loop(0, bpw, 8, unroll=4)
            def tbody(i):
                for u in range(8):
                    bi = i + u
                    bis = lane * 0 + bi
                    for q in range(d // _LANES):
                        vals = raw[bi, pl.ds(q * _LANES, _LANES)]
                        plsc.store_scatter(
                            tbuf, [dt_idx[q], di_idx, bis], vals)

        for s in range(_NBUF):
            gather(s, s).start()
        for s in range(_NBUF):
            gather(s, s).wait()
            transpose(s)
            write(s, s).start()
            gather(s, s + _NBUF).start()

        def ring(i, carry):
            j = i * _NBUF
            for s in range(_NBUF):
                c = j + s
                gather(s, c).wait()
                write(s, c - _NBUF).wait()
                transpose(s)
                write(s, c).start()
                gather(s, c + _NBUF).start()
            return carry

        lax.fori_loop(1, l_ // _NBUF - 1, ring, 0)

        tail = l_ - _NBUF
        for s in range(_NBUF):
            c = tail + s
            gather(s, c).wait()
            write(s, c - _NBUF).wait()
            transpose(s)
            write(s, c).start()
        for s in range(_NBUF):
            write(s, tail + s).wait()

    return gather_kernel


def kernel(word_ids, emb_weight):
    b_, l_ = word_ids.shape
    _, d = emb_weight.shape
    info = plsc.get_sparse_core_info()
    nw = info.num_cores * info.num_subcores
    assert b_ % nw == 0 and l_ % _NBUF == 0 and d % _LANES == 0
    out6 = _build(b_, l_, d, nw, info.num_cores)(emb_weight, word_ids.T)
    return out6.transpose(2, 4, 0, 1, 3).reshape(b_, l_, d)


# speedup vs baseline: 1.5305x; 1.5305x over previous
import functools

import jax
import jax.numpy as jnp
from jax import lax
from jax.experimental import pallas as pl
from jax.experimental.pallas import tpu as pltpu
from jax.experimental.pallas import tpu_sc as plsc

_NBUF = 5
_LANES = 16
_TPAD = 136


@functools.cache
def _build(b_, l_, d, nw, nc):
    bpw = b_ // nw
    dt_n = d // 8
    mesh = plsc.VectorSubcoreMesh(core_axis_name="c", subcore_axis_name="s")

    @functools.partial(
        pl.kernel,
        mesh=mesh,
        out_type=jax.ShapeDtypeStruct((l_, dt_n, b_ // bpw, 8, bpw), jnp.float32),
        scratch_types=[
            pltpu.VMEM((l_, bpw), jnp.int32),
            *[pltpu.VMEM((bpw, d), jnp.float32) for _ in range(_NBUF)],
            *[pltpu.VMEM((dt_n, 8, _TPAD), jnp.float32) for _ in range(_NBUF)],
            pltpu.SemaphoreType.DMA((_NBUF,)),
            pltpu.SemaphoreType.DMA((_NBUF,)),
        ],
        compiler_params=pltpu.CompilerParams(
            use_tc_tiling_on_sc=False, needs_layout_passes=False),
    )
    def gather_kernel(table, idx_t, out, idx_v, *bufs):
        raws = bufs[:_NBUF]
        tbufs = bufs[_NBUF:2 * _NBUF]
        gsem, osem = bufs[2 * _NBUF], bufs[2 * _NBUF + 1]
        wid = lax.axis_index("s") * nc + lax.axis_index("c")
        b0 = wid * bpw
        pltpu.sync_copy(idx_t.at[:, pl.ds(b0, bpw)], idx_v)

        lane = lax.iota(jnp.int32, _LANES)
        di_idx = lane & 7
        dt_idx = [(lane >> 3) + 2 * q for q in range(d // _LANES)]

        def gather(s, c):
            return pltpu.make_async_copy(
                table.at[idx_v.at[c]], raws[s], gsem.at[s])

        def write(s, c):
            return pltpu.make_async_copy(
                tbufs[s].at[:, :, pl.ds(0, bpw)], out.at[c, :, wid],
                osem.at[s])

        def transpose(s):
            raw, tbuf = raws[s], tbufs[s]

            @plsc.parallel_loop(0, bpw, 4, unroll=2)
            def tbody(i):
                for u in range(4):
                    bi = i + u
                    bis = lane * 0 + bi
                    for q in range(d // _LANES):
                        vals = raw[bi, pl.ds(q * _LANES, _LANES)]
                        plsc.store_scatter(
                            tbuf, [dt_idx[q], di_idx, bis], vals)

        for s in range(_NBUF):
            gather(s, s).start()
        for s in range(_NBUF):
            gather(s, s).wait()
            transpose(s)
            write(s, s).start()
            gather(s, s + _NBUF).start()

        def ring(i, carry):
            j = i * _NBUF
            for s in range(_NBUF):
                c = j + s
                gather(s, c).wait()
                write(s, c - _NBUF).wait()
                transpose(s)
                write(s, c).start()
                gather(s, c + _NBUF).start()
            return carry

        lax.fori_loop(1, l_ // _NBUF - 1, ring, 0)

        tail = l_ - _NBUF
        for s in range(_NBUF):
            c = tail + s
            gather(s, c).wait()
            write(s, c - _NBUF).wait()
            transpose(s)
            write(s, c).start()
        for s in range(_NBUF):
            write(s, tail + s).wait()

    return gather_kernel


def kernel(word_ids, emb_weight):
    b_, l_ = word_ids.shape
    _, d = emb_weight.shape
    info = plsc.get_sparse_core_info()
    nw = info.num_cores * info.num_subcores
    assert b_ % nw == 0 and l_ % _NBUF == 0 and d % _LANES == 0
    out6 = _build(b_, l_, d, nw, info.num_cores)(emb_weight, word_ids.T)
    return out6.transpose(2, 4, 0, 1, 3).reshape(b_, l_, d)
